# Initial kernel scaffold; baseline (speedup 1.0000x reference)
import jax
import jax.numpy as jnp
from jax.experimental import pallas as pl


def kernel(x_pano, x_footprint, edge_index_pano, edge_index_rev, params):
    raise NotImplementedError("write your pallas kernel here")


# speedup vs baseline: 8.9816x; 8.9816x over previous
import functools

import jax
import jax.numpy as jnp
from jax import lax
from jax.experimental import pallas as pl
from jax.experimental.pallas import tpu as pltpu
from jax.experimental.pallas import tpu_sc as plsc

_N = 10000
_D_IN = 128
_HID = 64
_NC = 2
_NS = 16
_NW = _NC * _NS
_B = 128
_NPAD = 10240
_RPT = _NPAD // _NS
_ACC_W = 80


def _round_up(x, m):
    return (x + m - 1) // m * m


@functools.lru_cache(maxsize=None)
def _make_edge_pass(e_pad):
    per_tile = e_pad // _NW
    nblk = per_tile // _B
    mesh = plsc.VectorSubcoreMesh(core_axis_name="c", subcore_axis_name="s")

    @functools.partial(
        pl.kernel,
        mesh=mesh,
        compiler_params=pltpu.CompilerParams(use_tc_tiling_on_sc=False),
        out_type=jax.ShapeDtypeStruct((_NC, _NPAD, _ACC_W), jnp.float32),
        scratch_types=[
            pltpu.VMEM((_B,), jnp.int32),
            pltpu.VMEM((_B,), jnp.int32),
            pltpu.VMEM((_B, 64), jnp.float32),
            pltpu.VMEM((_B, 64), jnp.float32),
            pltpu.VMEM((_B, _ACC_W), jnp.float32),
            pltpu.VMEM((64,), jnp.float32),
            pltpu.VMEM_SHARED((_NPAD, _ACC_W), jnp.float32),
            pltpu.SemaphoreType.DMA,
        ],
    )
    def edge_pass(xl_hbm, xr_hbm, src_hbm, dst_hbm, att_hbm, out_hbm,
                  sidx, didx, xlv, xrv, msg, attv, acc, sem):
        cid = lax.axis_index("c")
        sid = lax.axis_index("s")
        wid = cid * _NS + sid

        zero16 = jnp.zeros((16,), jnp.float32)

        def zrow(i, carry):
            for k in range(_ACC_W // 16):
                msg[i, pl.ds(16 * k, 16)] = zero16
            return carry

        lax.fori_loop(0, _B, zrow, 0)
        row0 = sid * _RPT
        done = 0
        while done < _RPT:
            n = min(_B, _RPT - done)
            pltpu.sync_copy(msg.at[pl.ds(0, n)], acc.at[pl.ds(row0 + done, n)])
            done += n
        pltpu.sync_copy(att_hbm, attv)
        plsc.subcore_barrier()

        a_ch = [attv[pl.ds(16 * k, 16)] for k in range(4)]
        base = wid * per_tile
        lane = lax.iota(jnp.int32, 16)

        def hsum(v):
            for sh in (8, 4, 2, 1):
                v = v + v.at[lane ^ sh].get(mode='promise_in_bounds')
            return v

        def block_body(b, carry):
            off = pl.multiple_of(base + b * _B, 8)
            pltpu.sync_copy(src_hbm.at[pl.ds(off, _B)], sidx)
            pltpu.sync_copy(dst_hbm.at[pl.ds(off, _B)], didx)
            pltpu.async_copy(xl_hbm.at[sidx], xlv, sem).wait()
            pltpu.async_copy(xr_hbm.at[didx], xrv, sem).wait()

            def edge_body(e, ecarry):
                zl = [xlv[e, pl.ds(16 * k, 16)] for k in range(4)]
                accv = zero16
                for k in range(4):
                    z = zl[k] + xrv[e, pl.ds(16 * k, 16)]
                    lr = jnp.maximum(z, 0.2 * z)
                    accv = accv + a_ch[k] * lr
                w = jnp.exp(hsum(accv))
                for k in range(4):
                    msg[e, pl.ds(16 * k, 16)] = zl[k] * w
                msg[e, pl.ds(64, 16)] = w
                return ecarry

            lax.fori_loop(0, _B, edge_body, 0)
            pltpu.sync_copy(msg, acc.at[didx], add=True)
            return carry

        lax.fori_loop(0, nblk, block_body, 0)
        plsc.subcore_barrier()

        done = 0
        while done < _RPT:
            n = min(_B, _RPT - done)
            pltpu.sync_copy(acc.at[pl.ds(row0 + done, n)],
                            out_hbm.at[cid, pl.ds(row0 + done, n)])
            done += n

    return edge_pass


def _pad_edges(src, dst, e_raw):
    e_pad = _round_up(e_raw, _NW * _B)
    pad = e_pad - e_raw
    src = jnp.concatenate([src, jnp.zeros((pad,), jnp.int32)])
    dst = jnp.concatenate([dst, jnp.full((pad,), _N, jnp.int32)])
    return src, dst, e_pad


def _mm(x, w, b):
    return jnp.dot(x, w, preferred_element_type=jnp.float32) + b


def _proj_kernel(x_ref, wl_ref, bl_ref, wr_ref, br_ref, xl_ref, xr_ref):
    x = x_ref[...]
    xl_ref[...] = _mm(x, wl_ref[...], bl_ref[...])
    xr_ref[...] = _mm(x, wr_ref[...], br_ref[...])


def _proj(x, wl, bl, wr, br):
    outs = [jax.ShapeDtypeStruct((x.shape[0], wl.shape[1]), jnp.float32)] * 2
    return pl.pallas_call(_proj_kernel, out_shape=outs)(
        x, wl, bl.reshape(1, -1), wr, br.reshape(1, -1))


def _norm(acc, bias):
    t = acc[0] + acc[1]
    num = t[:_N, :64]
    den = t[:_N, 64:65]
    return num / (den + 1e-16) + bias


def _combine_proj_kernel(acc_ref, bias_ref, wl_ref, bl_ref, wr_ref, br_ref,
                         xl_ref, xr_ref):
    h = _norm(acc_ref[...], bias_ref[...])
    xl_ref[...] = _mm(h, wl_ref[...], bl_ref[...])
    xr_ref[...] = _mm(h, wr_ref[...], br_ref[...])


def _combine_proj(acc, bias, wl, bl, wr, br):
    outs = [jax.ShapeDtypeStruct((_N, wl.shape[1]), jnp.float32)] * 2
    return pl.pallas_call(_combine_proj_kernel, out_shape=outs)(
        acc, bias.reshape(1, -1), wl, bl.reshape(1, -1), wr, br.reshape(1, -1))


def _mixed_proj_kernel(acc_ref, bias_ref, wl_ref, bl_ref, xf_ref, wr_ref,
                       br_ref, xl_ref, xr_ref):
    h = _norm(acc_ref[...], bias_ref[...])
    xl_ref[...] = _mm(h, wl_ref[...], bl_ref[...])
    xr_ref[...] = _mm(xf_ref[...], wr_ref[...], br_ref[...])


def _mixed_proj(acc, bias, wl, bl, xf, wr, br):
    outs = [jax.ShapeDtypeStruct((_N, wl.shape[1]), jnp.float32)] * 2
    return pl.pallas_call(_mixed_proj_kernel, out_shape=outs)(
        acc, bias.reshape(1, -1), wl, bl.reshape(1, -1), xf, wr,
        br.reshape(1, -1))


def _final_kernel(acc_ref, bias_ref, xf_ref, p_refs, out_ref):
    foot = _norm(acc_ref[...], bias_ref[...])
    p = {k: r[...] for k, r in p_refs.items()}
    m = jax.nn.relu(_mm(foot, p['mlp_W1'], p['mlp_b1']))
    m = jax.nn.relu(_mm(m, p['mlp_W2'], p['mlp_b2']))
    m = _mm(m, p['mlp_W3'], p['mlp_b3'])
    xf = xf_ref[...]
    lin = _mm(xf, p['null_W_lin'], p['null_b_lin'])
    a = jax.nn.relu(_mm(xf, p['null_W_s'], p['null_b_s']))
    a = jax.nn.relu(_mm(a, p['null_W_c'], p['null_b_c']))
    a = jax.nn.relu(_mm(a, p['null_W_c'], p['null_b_c']))
    a = _mm(a, p['null_W_e'], p['null_b_e'])
    out_ref[...] = lin + a + m


def _final(acc, bias, xf, params):
    keys = ['mlp_W1', 'mlp_b1', 'mlp_W2', 'mlp_b2', 'mlp_W3', 'mlp_b3',
            'null_W_lin', 'null_b_lin', 'null_W_s', 'null_b_s',
            'null_W_c', 'null_b_c', 'null_W_e', 'null_b_e']
    p = {k: (params[k].reshape(1, -1) if params[k].ndim == 1 else params[k])
         for k in keys}
    return pl.pallas_call(
        _final_kernel,
        out_shape=jax.ShapeDtypeStruct((_N, 1), jnp.float32),
    )(acc, bias.reshape(1, -1), xf, p)


def kernel(x_pano, x_footprint, edge_index_pano, edge_index_rev, params):
    p = params
    src_p, dst_p, e_pano = _pad_edges(
        edge_index_pano[0], edge_index_pano[1], edge_index_pano.shape[1])
    src_r, dst_r, e_rev = _pad_edges(
        edge_index_rev[0], edge_index_rev[1], edge_index_rev.shape[1])
    pano_pass = _make_edge_pass(e_pano)
    rev_pass = _make_edge_pass(e_rev)

    xl0, xr0 = _proj(x_pano, p['conv0_Wl'], p['conv0_bl'],
                     p['conv0_Wr'], p['conv0_br'])
    acc0 = pano_pass(xl0, xr0, src_p, dst_p, p['conv0_att'].reshape(64))

    xl1, xr1 = _combine_proj(acc0, p['conv0_bias'], p['conv1_Wl'],
                             p['conv1_bl'], p['conv1_Wr'], p['conv1_br'])
    acc1 = pano_pass(xl1, xr1, src_p, dst_p, p['conv1_att'].reshape(64))

    xlt, xrt = _mixed_proj(acc1, p['conv1_bias'], p['convt_Wl'],
                           p['convt_bl'], x_footprint, p['convt_Wr'],
                           p['convt_br'])
    acct = rev_pass(xlt, xrt, src_r, dst_r, p['convt_att'].reshape(64))

    return _final(acct, p['convt_bias'], x_footprint, params)
